# baseline (speedup 1.0000x reference)
import math

import jax
import jax.numpy as jnp
from jax import lax
from jax.experimental import pallas as pl
from jax.experimental.pallas import tpu as pltpu
from jax.experimental.pallas import tpu_sc as plsc

S = 65536
T = 65536
BR = 32
B = 16
L = 16
BH = 8

LN_SMOOTH = math.log(0.9)

NC, NS = 2, 16
SRC_PER_TILE = S // NS
CHUNK = 128
N_CHUNKS = SRC_PER_TILE // CHUNK
WROWS = CHUNK * BR // 128
CROWS = CHUNK * BR
T_PER_TILE = T // NS


def _sc_body(spikes, att, dly, tgt, zrows, out, sp_buf, a_buf, d_buf,
             tgt_buf, contrib, acc, sem_in, sem_sc):
    cid = lax.axis_index("c")
    sid = lax.axis_index("s")

    pltpu.sync_copy(zrows, acc.at[pl.ds(sid * T_PER_TILE, T_PER_TILE)])
    plsc.subcore_barrier()

    iota16 = lax.iota(jnp.int32, L)
    half8 = jnp.bitwise_and(iota16, 7)
    hi = jnp.right_shift(iota16, 3)
    pair = [2 * q + hi for q in range(16)]
    pair2d = [p[:, None] for p in pair[:8]]
    dnums = lax.GatherDimensionNumbers(
        offset_dims=(), collapsed_slice_dims=(0,), start_index_map=(0,))

    def bcast_pair(vec, q):
        return lax.gather(vec, pair2d[q], dnums, (1,),
                          mode=lax.GatherScatterMode.PROMISE_IN_BOUNDS)

    def in_slices(i):
        p2 = jnp.bitwise_and(i, 1)
        p3 = lax.rem(i, 3)
        src0 = pl.multiple_of(sid * SRC_PER_TILE + i * CHUNK, CHUNK)
        row0 = pl.multiple_of(src0 // 4, CHUNK // 4)
        return ((spikes.at[pl.ds(cid * BH, BH), pl.ds(src0, CHUNK)],
                 sp_buf.at[p2, :, pl.ds(0, CHUNK)]),
                (att.at[pl.ds(src0, CHUNK)], a_buf.at[p2]),
                (dly.at[pl.ds(src0, CHUNK)], d_buf.at[p2]),
                (tgt.at[pl.ds(row0, WROWS)], tgt_buf.at[p3]))

    def fire_inputs(i):
        for src, dst in in_slices(i):
            pltpu.async_copy(src, dst, sem_in)

    def wait_inputs(i):
        for src, dst in in_slices(i):
            pltpu.make_async_copy(src, dst, sem_in).wait()

    def trow(p3, j):
        return tgt_buf.at[p3, j]

    def scat_desc(p2, p3, j):
        j128 = pl.multiple_of(j * 128, 128)
        return pltpu.make_async_copy(contrib.at[p2, pl.ds(j128, 128)],
                                     acc.at[trow(p3, j)], sem_sc)

    fire_inputs(0)

    def chunk_body(i, _):
        p2 = jnp.bitwise_and(i, 1)
        p3 = lax.rem(i, 3)
        wait_inputs(i)

        @pl.when(i >= 2)
        def _():
            p3d = lax.rem(i + 1, 3)

            def drain_body(j, _):
                scat_desc(p2, p3d, j).wait()
                return 0

            lax.fori_loop(0, WROWS, drain_body, 0)

        @pl.when(i + 1 < N_CHUNKS)
        def _():
            fire_inputs(i + 1)

        spb = sp_buf.at[p2]
        ab = a_buf.at[p2]
        db = d_buf.at[p2]
        ctb = contrib.at[p2]

        def grp_body(j, _):
            for cc in range(4):
                c = j * 4 + cc
                spk = plsc.load_gather(spb, [half8,
                                             jnp.full((L,), c, jnp.int32)])
                a_lo = ab[c, pl.ds(0, 16)]
                a_hi = ab[c, pl.ds(16, 16)]
                d_lo = db[c, pl.ds(0, 16)].astype(jnp.float32)
                d_hi = db[c, pl.ds(16, 16)].astype(jnp.float32)
                w_lo = (jnp.clip(a_lo, 0.0, 1.0)
                        * jnp.exp(d_lo * LN_SMOOTH))
                w_hi = (jnp.clip(a_hi, 0.0, 1.0)
                        * jnp.exp(d_hi * LN_SMOOTH))
                rowb = jnp.full((L,), c * 32, jnp.int32)
                for q in range(8):
                    plsc.store_scatter(ctb, [rowb + pair[q], half8],
                                       spk * bcast_pair(w_lo, q))
                for q in range(8):
                    plsc.store_scatter(ctb, [rowb + pair[8 + q], half8],
                                       spk * bcast_pair(w_hi, q))
            return 0

        def grp_scat(j, _):
            grp_body(j, 0)
            j128 = pl.multiple_of(j * 128, 128)
            pltpu.async_copy(contrib.at[p2, pl.ds(j128, 128)],
                             acc.at[trow(p3, j)], sem_sc, add=True)
            return 0

        lax.fori_loop(0, WROWS, grp_scat, 0)
        return 0

    lax.fori_loop(0, N_CHUNKS, chunk_body, 0)

    for k in (N_CHUNKS - 2, N_CHUNKS - 1):

        def tail_drain(j, _, k=k):
            scat_desc(k & 1, k % 3, j).wait()
            return 0

        lax.fori_loop(0, WROWS, tail_drain, 0)

    plsc.subcore_barrier()
    pltpu.sync_copy(acc.at[pl.ds(sid * T_PER_TILE, T_PER_TILE)],
                    out.at[cid, pl.ds(sid * T_PER_TILE, T_PER_TILE)])


_sc_scatter = pl.kernel(
    _sc_body,
    out_type=jax.ShapeDtypeStruct((NC, T, BH), jnp.float32),
    mesh=plsc.VectorSubcoreMesh(core_axis_name="c", subcore_axis_name="s",
                                num_cores=NC, num_subcores=NS),
    scratch_types=[
        pltpu.VMEM((2, BH, 137), jnp.float32),
        pltpu.VMEM((2, CHUNK, BR), jnp.float32),
        pltpu.VMEM((2, CHUNK, BR), jnp.int32),
        pltpu.VMEM((3, WROWS, 128), jnp.int32),
        pltpu.VMEM((2, CROWS, BH), jnp.float32),
        pltpu.VMEM_SHARED((T, BH), jnp.float32),
        pltpu.SemaphoreType.DMA,
        pltpu.SemaphoreType.DMA,
    ],
    compiler_params=pltpu.CompilerParams(needs_layout_passes=False,
                                         use_tc_tiling_on_sc=False),
)


def _post_body(acc_ref, out_ref):
    out_ref[...] = jnp.concatenate([acc_ref[0].T, acc_ref[1].T], axis=0)


def _post(acc):
    blk = 4096
    return pl.pallas_call(
        _post_body,
        grid=(T // blk,),
        in_specs=[pl.BlockSpec((NC, blk, BH), lambda i: (0, i, 0))],
        out_specs=pl.BlockSpec((B, blk), lambda i: (0, i)),
        out_shape=jax.ShapeDtypeStruct((B, T), jnp.float32),
    )(acc)


def kernel(spikes, attenuation, target_indices, delays):
    tgt32 = target_indices.astype(jnp.int32).reshape(S * BR // 128, 128)
    zrows = jnp.zeros((T_PER_TILE, BH), jnp.float32)
    acc = _sc_scatter(spikes, attenuation, delays, tgt32, zrows)
    return _post(acc)


# speedup vs baseline: 1.3695x; 1.3695x over previous
import math

import jax
import jax.numpy as jnp
from jax import lax
from jax.experimental import pallas as pl
from jax.experimental.pallas import tpu as pltpu
from jax.experimental.pallas import tpu_sc as plsc

S = 65536
T = 65536
BR = 32
B = 16
L = 16
BH = 8

LN_SMOOTH = math.log(0.9)

NC, NS = 2, 16
SRC_PER_TILE = S // NS
CHUNK = 128
N_CHUNKS = SRC_PER_TILE // CHUNK
WROWS = CHUNK * BR // 128
CROWS = CHUNK * BR
T_PER_TILE = T // NS
TB = 1024


def _pre_body(att_ref, dly_ref, w_ref):
    att = jnp.clip(att_ref[...], 0.0, 1.0)
    decay = jnp.exp(dly_ref[...].astype(jnp.float32) * LN_SMOOTH)
    w_ref[...] = att * decay


def _pre(att, dly):
    blk = 4096
    return pl.pallas_call(
        _pre_body,
        grid=(S // blk,),
        in_specs=[pl.BlockSpec((blk, BR), lambda i: (i, 0)),
                  pl.BlockSpec((blk, BR), lambda i: (i, 0))],
        out_specs=pl.BlockSpec((blk, BR), lambda i: (i, 0)),
        out_shape=jax.ShapeDtypeStruct((S, BR), jnp.float32),
    )(att, dly)


def _sc_body(spikes, w2, tgt2, zrows, out, sp_buf, w_buf, tgt_buf, contrib,
             tb, acc, sem_in, sem_sc):
    cid = lax.axis_index("c")
    sid = lax.axis_index("s")

    pltpu.sync_copy(zrows, acc.at[pl.ds(sid * T_PER_TILE, T_PER_TILE)])
    plsc.subcore_barrier()

    iota16 = lax.iota(jnp.int32, L)
    half8 = jnp.bitwise_and(iota16, 7)
    hi = jnp.right_shift(iota16, 3)
    pair = [2 * q + hi for q in range(16)]
    pair2d = [p[:, None] for p in pair[:8]]
    dnums = lax.GatherDimensionNumbers(
        offset_dims=(), collapsed_slice_dims=(0,), start_index_map=(0,))

    def bcast_pair(vec, q):
        return lax.gather(vec, pair2d[q], dnums, (1,),
                          mode=lax.GatherScatterMode.PROMISE_IN_BOUNDS)

    def in_slices(i):
        p2 = jnp.bitwise_and(i, 1)
        p3 = lax.rem(i, 3)
        src0 = pl.multiple_of(sid * SRC_PER_TILE + i * CHUNK, CHUNK)
        row0 = pl.multiple_of(src0 // 4, CHUNK // 4)
        return ((spikes.at[pl.ds(cid * BH, BH), pl.ds(src0, CHUNK)],
                 sp_buf.at[p2, :, pl.ds(0, CHUNK)]),
                (w2.at[pl.ds(row0, WROWS)], w_buf.at[p2]),
                (tgt2.at[pl.ds(row0, WROWS)], tgt_buf.at[p3]))

    def fire_inputs(i):
        for src, dst in in_slices(i):
            pltpu.async_copy(src, dst, sem_in)

    def wait_inputs(i):
        for src, dst in in_slices(i):
            pltpu.make_async_copy(src, dst, sem_in).wait()

    def scat_desc(p2, p3, j):
        j128 = pl.multiple_of(j * 128, 128)
        return pltpu.make_async_copy(contrib.at[p2, pl.ds(j128, 128)],
                                     acc.at[tgt_buf.at[p3, j]], sem_sc)

    fire_inputs(0)

    def chunk_body(i, _):
        p2 = jnp.bitwise_and(i, 1)
        p3 = lax.rem(i, 3)
        wait_inputs(i)

        @pl.when(i >= 2)
        def _():
            p3d = lax.rem(i + 1, 3)

            def drain_body(j, _):
                scat_desc(p2, p3d, j).wait()
                return 0

            lax.fori_loop(0, WROWS, drain_body, 0)

        @pl.when(i + 1 < N_CHUNKS)
        def _():
            fire_inputs(i + 1)

        spb = sp_buf.at[p2]
        wb = w_buf.at[p2]
        ctb = contrib.at[p2]

        def grp_body(j, _):
            for cc in range(4):
                c = j * 4 + cc
                spk = plsc.load_gather(spb, [half8,
                                             jnp.full((L,), c, jnp.int32)])
                w_lo = wb[j, pl.ds(cc * 32, 16)]
                w_hi = wb[j, pl.ds(cc * 32 + 16, 16)]
                cs = ctb.at[pl.ds(pl.multiple_of(c * 32, 32), 32)]
                for q in range(8):
                    plsc.store_scatter(cs, [pair[q], half8],
                                       spk * bcast_pair(w_lo, q))
                for q in range(8):
                    plsc.store_scatter(cs, [pair[8 + q], half8],
                                       spk * bcast_pair(w_hi, q))
            return 0

        def grp_scat(j, _):
            grp_body(j, 0)
            j128 = pl.multiple_of(j * 128, 128)
            pltpu.async_copy(contrib.at[p2, pl.ds(j128, 128)],
                             acc.at[tgt_buf.at[p3, j]], sem_sc, add=True)
            return 0

        lax.fori_loop(0, WROWS, grp_scat, 0)
        return 0

    lax.fori_loop(0, N_CHUNKS, chunk_body, 0)

    for k in (N_CHUNKS - 2, N_CHUNKS - 1):

        def tail_drain(j, _, k=k):
            scat_desc(k & 1, k % 3, j).wait()
            return 0

        lax.fori_loop(0, WROWS, tail_drain, 0)

    plsc.subcore_barrier()
    va = contrib.at[0]
    pltpu.sync_copy(acc.at[pl.ds(sid * T_PER_TILE, T_PER_TILE)], va)
    for p in range(T_PER_TILE // TB):

        def tp_body(t2, _, p=p):
            row = jnp.full((L,), p * TB + 2 * t2, jnp.int32) + hi
            v = plsc.load_gather(va, [row, half8])
            plsc.store_scatter(tb, [half8,
                                    jnp.full((L,), 2 * t2, jnp.int32) + hi],
                               v)
            return 0

        lax.fori_loop(0, TB // 2, tp_body, 0)
        pltpu.sync_copy(
            tb.at[:, pl.ds(0, TB)],
            out.at[cid, :, pl.ds(sid * T_PER_TILE + p * TB, TB)])


_sc_scatter = pl.kernel(
    _sc_body,
    out_type=jax.ShapeDtypeStruct((NC, BH, T), jnp.float32),
    mesh=plsc.VectorSubcoreMesh(core_axis_name="c", subcore_axis_name="s",
                                num_cores=NC, num_subcores=NS),
    scratch_types=[
        pltpu.VMEM((2, BH, 137), jnp.float32),
        pltpu.VMEM((2, WROWS, 128), jnp.float32),
        pltpu.VMEM((3, WROWS, 128), jnp.int32),
        pltpu.VMEM((2, CROWS, BH), jnp.float32),
        pltpu.VMEM((BH, TB + 3), jnp.float32),
        pltpu.VMEM_SHARED((T, BH), jnp.float32),
        pltpu.SemaphoreType.DMA,
        pltpu.SemaphoreType.DMA,
    ],
    compiler_params=pltpu.CompilerParams(needs_layout_passes=False,
                                         use_tc_tiling_on_sc=False),
)


def kernel(spikes, attenuation, target_indices, delays):
    w2 = _pre(attenuation, delays).reshape(S * BR // 128, 128)
    tgt2 = target_indices.astype(jnp.int32).reshape(S * BR // 128, 128)
    zrows = jnp.zeros((T_PER_TILE, BH), jnp.float32)
    return _sc_scatter(spikes, w2, tgt2, zrows).reshape(B, T)
